# baseline (speedup 1.0000x reference)
import functools

import jax
import jax.numpy as jnp
from jax import lax
from jax.experimental import pallas as pl
from jax.experimental.pallas import tpu as pltpu
from jax.experimental.pallas import tpu_sc as plsc

LENGTH = 1000000
DIM = 64
BATCH = 16384

NUM_CORES = 2
NUM_SUBCORES = 16
NUM_WORKERS = NUM_CORES * NUM_SUBCORES
B_PER_W = BATCH // NUM_WORKERS
IDX_CHUNK = 128
N_CHUNKS = B_PER_W // IDX_CHUNK
LANES = 16


def _sc_gather_body(xs_hbm, ti_hbm, tg_hbm, out_in_hbm, out_gt_hbm,
                    xs_v, idx_v, rows_in_v, rows_gt_v, sem):
    wid = lax.axis_index("s") * NUM_CORES + lax.axis_index("c")
    base = wid * B_PER_W

    pltpu.sync_copy(xs_hbm.at[pl.ds(base, B_PER_W)], xs_v)

    scale = jnp.float32(LENGTH)
    for j in range(N_CHUNKS):
        for c in range(IDX_CHUNK // LANES):
            off = j * IDX_CHUNK + c * LANES
            v = xs_v[pl.ds(off, LANES)] * scale
            iv = v.astype(jnp.int32)
            iv = jnp.minimum(jnp.maximum(iv, 0), LENGTH - 1)
            idx_v[j, pl.ds(c * LANES, LANES)] = iv

    copies = []
    for j in range(N_CHUNKS):
        copies.append(pltpu.async_copy(
            ti_hbm.at[idx_v.at[j]],
            rows_in_v.at[pl.ds(j * IDX_CHUNK, IDX_CHUNK)], sem))
        copies.append(pltpu.async_copy(
            tg_hbm.at[idx_v.at[j]],
            rows_gt_v.at[pl.ds(j * IDX_CHUNK, IDX_CHUNK)], sem))
    for cp in copies:
        cp.wait()

    pltpu.sync_copy(rows_in_v, out_in_hbm.at[pl.ds(base, B_PER_W)])
    pltpu.sync_copy(rows_gt_v, out_gt_hbm.at[pl.ds(base, B_PER_W)])


@jax.jit
def kernel(xs, table_input, table_gt):
    mesh = plsc.VectorSubcoreMesh(core_axis_name="c", subcore_axis_name="s")
    run = functools.partial(
        pl.kernel,
        out_type=(
            jax.ShapeDtypeStruct((BATCH, DIM), jnp.float32),
            jax.ShapeDtypeStruct((BATCH, DIM), jnp.float32),
        ),
        mesh=mesh,
        scratch_types=[
            pltpu.VMEM((B_PER_W,), jnp.float32),
            pltpu.VMEM((N_CHUNKS, IDX_CHUNK), jnp.int32),
            pltpu.VMEM((B_PER_W, DIM), jnp.float32),
            pltpu.VMEM((B_PER_W, DIM), jnp.float32),
            pltpu.SemaphoreType.DMA,
        ],
        compiler_params=pltpu.CompilerParams(use_tc_tiling_on_sc=False),
    )(_sc_gather_body)
    return run(xs, table_input, table_gt)


# speedup vs baseline: 1.5825x; 1.5825x over previous
import functools

import jax
import jax.numpy as jnp
from jax import lax
from jax.experimental import pallas as pl
from jax.experimental.pallas import tpu as pltpu
from jax.experimental.pallas import tpu_sc as plsc

LENGTH = 1000000
DIM = 64
BATCH = 16384

NUM_CORES = 2
NUM_SUBCORES = 16
NUM_WORKERS = NUM_CORES * NUM_SUBCORES
B_PER_W = BATCH // NUM_WORKERS
LANES = 16


def _sc_gather_body(xs_hbm, ti_hbm, tg_hbm, out_in_hbm, out_gt_hbm,
                    xs_v, idx_v, rows_v, sem):
    wid = lax.axis_index("s") * NUM_CORES + lax.axis_index("c")
    base = wid * B_PER_W

    pltpu.sync_copy(xs_hbm.at[pl.ds(base, B_PER_W)], xs_v)

    scale = jnp.float32(LENGTH)
    for j in range(B_PER_W // LANES):
        v = xs_v[pl.ds(j * LANES, LANES)] * scale
        iv = v.astype(jnp.int32)
        iv = jnp.minimum(jnp.maximum(iv, 0), LENGTH - 1)
        idx_v[pl.ds(j * LANES, LANES)] = iv

    def gather_pass(table_hbm, out_hbm):
        def issue(j, carry):
            off = pl.multiple_of(j * LANES, LANES)
            iv = idx_v[pl.ds(off, LANES)]
            for l in range(LANES):
                r = iv[l]
                i = off + l
                pltpu.async_copy(table_hbm.at[pl.ds(r, 1)],
                                 rows_v.at[pl.ds(i, 1)], sem)
            return carry

        lax.fori_loop(0, B_PER_W // LANES, issue, 0)

        def drain(i, carry):
            pltpu.make_async_copy(table_hbm.at[pl.ds(0, 1)],
                                  rows_v.at[pl.ds(i, 1)], sem).wait()
            return carry

        lax.fori_loop(0, B_PER_W, drain, 0)

        pltpu.sync_copy(rows_v, out_hbm.at[pl.ds(base, B_PER_W)])

    gather_pass(ti_hbm, out_in_hbm)
    gather_pass(tg_hbm, out_gt_hbm)


@jax.jit
def kernel(xs, table_input, table_gt):
    mesh = plsc.VectorSubcoreMesh(core_axis_name="c", subcore_axis_name="s")
    run = functools.partial(
        pl.kernel,
        out_type=(
            jax.ShapeDtypeStruct((BATCH, DIM), jnp.float32),
            jax.ShapeDtypeStruct((BATCH, DIM), jnp.float32),
        ),
        mesh=mesh,
        scratch_types=[
            pltpu.VMEM((B_PER_W,), jnp.float32),
            pltpu.VMEM((B_PER_W,), jnp.int32),
            pltpu.VMEM((B_PER_W, DIM), jnp.float32),
            pltpu.SemaphoreType.DMA,
        ],
    )(_sc_gather_body)
    return run(xs, table_input, table_gt)
